# baseline (speedup 1.0000x reference)
import jax
import jax.numpy as jnp
from jax import lax
from jax.experimental import pallas as pl
from jax.experimental.pallas import tpu as pltpu
from jax.experimental.pallas import tpu_sc as plsc
from jax._src.pallas import mpmd as _mpmd

_B = 4096
_H = 16384
_NW = 32
_RPW = _B // _NW
_BR = 256
_NCH = _B // _BR
_FDEPTH = 4
_SDEPTH = 4


def _fill_body(slot_hbm, out_hbm, zbuf, sems):
    del slot_hbm
    zbuf[...] = jnp.zeros((_BR, _H), jnp.float32)

    def mk(g):
        return pltpu.make_async_copy(
            zbuf, out_hbm.at[pl.ds(g * _BR, _BR), :], sems.at[g % _FDEPTH]
        )

    for g in range(_NCH):
        if g >= _FDEPTH:
            mk(g - _FDEPTH).wait()
        mk(g).start()
    for g in range(_NCH - _FDEPTH, _NCH):
        mk(g).wait()


def _poke_body(filled_hbm, slot_hbm, out_hbm, slot_v, seg_v, sems):
    del filled_hbm
    nc = 2
    wid = lax.axis_index("s") * nc + lax.axis_index("c")
    base = wid * _RPW

    pltpu.sync_copy(slot_hbm.at[pl.ds(base, _RPW)], slot_v)

    lane = lax.iota(jnp.int32, 16)

    def group(g, _):
        sv = slot_v[pl.ds(g * 16, 16)]
        for k in range(16):
            d = k % _SDEPTH
            r = g * 16 + k
            s = sv[k]
            seg = lax.div(s, 16)
            within = s - seg * 16

            def _wait_prev():
                pltpu.make_async_copy(
                    seg_v.at[d],
                    out_hbm.at[base + r].at[pl.ds(seg * 16, 16)],
                    sems.at[d],
                ).wait()

            if k >= _SDEPTH:
                _wait_prev()
            else:
                pl.when(g > 0)(_wait_prev)

            seg_v[d] = (lane == within).astype(jnp.float32)
            pltpu.make_async_copy(
                seg_v.at[d],
                out_hbm.at[base + r].at[pl.ds(seg * 16, 16)],
                sems.at[d],
            ).start()
        return 0

    lax.fori_loop(0, _RPW // 16, group, 0)
    for d in range(_SDEPTH):
        pltpu.make_async_copy(
            seg_v.at[d], out_hbm.at[base].at[pl.ds(0, 16)], sems.at[d]
        ).wait()


def kernel(hidden_activation, slot_i):
    b, h = hidden_activation.shape
    filled = pl.pallas_call(
        _fill_body,
        in_specs=[pl.BlockSpec(memory_space=pltpu.MemorySpace.HBM)],
        out_specs=pl.BlockSpec(memory_space=pltpu.MemorySpace.HBM),
        out_shape=jax.ShapeDtypeStruct((b, h), jnp.float32),
        scratch_shapes=[
            pltpu.VMEM((_BR, _H), jnp.float32),
            pltpu.SemaphoreType.DMA((_FDEPTH,)),
        ],
    )(slot_i)

    mesh = plsc.VectorSubcoreMesh(core_axis_name="c", subcore_axis_name="s")
    out = _mpmd._mpmd_map(
        [(mesh, _poke_body)],
        jax.ShapeDtypeStruct((b, h), jnp.float32),
        input_output_aliases={0: 0},
        compiler_params=pltpu.CompilerParams(needs_layout_passes=False),
        scratch_types=[
            pltpu.VMEM((_RPW,), jnp.int32),
            pltpu.VMEM((_SDEPTH, 16), jnp.float32),
            pltpu.SemaphoreType.DMA((_SDEPTH,)),
        ],
    )(filled, slot_i)
    return out


# speedup vs baseline: 1.0107x; 1.0107x over previous
import jax
import jax.numpy as jnp
from jax import lax
from jax.experimental import pallas as pl
from jax.experimental.pallas import tpu as pltpu
from jax.experimental.pallas import tpu_sc as plsc
from jax._src.pallas import mpmd as _mpmd

_B = 4096
_H = 16384
_NW = 32
_RPW = _B // _NW
_BR = 64
_NCH = _B // _BR
_FDEPTH = 8
_SDEPTH = 4


def _fill_body(slot_hbm, out_hbm, zbuf, sems):
    del slot_hbm
    zbuf[...] = jnp.zeros((_BR, _H), jnp.float32)

    def mk(g):
        return pltpu.make_async_copy(
            zbuf, out_hbm.at[pl.ds(g * _BR, _BR), :], sems.at[g % _FDEPTH]
        )

    for g in range(_NCH):
        if g >= _FDEPTH:
            mk(g - _FDEPTH).wait()
        mk(g).start()
    for g in range(_NCH - _FDEPTH, _NCH):
        mk(g).wait()


def _poke_body(filled_hbm, slot_hbm, out_hbm, slot_v, seg_v, sems):
    del filled_hbm
    nc = 2
    wid = lax.axis_index("s") * nc + lax.axis_index("c")
    base = wid * _RPW

    pltpu.sync_copy(slot_hbm.at[pl.ds(base, _RPW)], slot_v)

    lane = lax.iota(jnp.int32, 16)

    def group(g, _):
        sv = slot_v[pl.ds(g * 16, 16)]
        for k in range(16):
            d = k % _SDEPTH
            r = g * 16 + k
            s = sv[k]
            seg = lax.div(s, 16)
            within = s - seg * 16

            def _wait_prev():
                pltpu.make_async_copy(
                    seg_v.at[d],
                    out_hbm.at[base + r].at[pl.ds(seg * 16, 16)],
                    sems.at[d],
                ).wait()

            if k >= _SDEPTH:
                _wait_prev()
            else:
                pl.when(g > 0)(_wait_prev)

            seg_v[d] = (lane == within).astype(jnp.float32)
            pltpu.make_async_copy(
                seg_v.at[d],
                out_hbm.at[base + r].at[pl.ds(seg * 16, 16)],
                sems.at[d],
            ).start()
        return 0

    lax.fori_loop(0, _RPW // 16, group, 0)
    for d in range(_SDEPTH):
        pltpu.make_async_copy(
            seg_v.at[d], out_hbm.at[base].at[pl.ds(0, 16)], sems.at[d]
        ).wait()


def kernel(hidden_activation, slot_i):
    b, h = hidden_activation.shape
    filled = pl.pallas_call(
        _fill_body,
        in_specs=[pl.BlockSpec(memory_space=pltpu.MemorySpace.HBM)],
        out_specs=pl.BlockSpec(memory_space=pltpu.MemorySpace.HBM),
        out_shape=jax.ShapeDtypeStruct((b, h), jnp.float32),
        scratch_shapes=[
            pltpu.VMEM((_BR, _H), jnp.float32),
            pltpu.SemaphoreType.DMA((_FDEPTH,)),
        ],
    )(slot_i)

    mesh = plsc.VectorSubcoreMesh(core_axis_name="c", subcore_axis_name="s")
    out = _mpmd._mpmd_map(
        [(mesh, _poke_body)],
        jax.ShapeDtypeStruct((b, h), jnp.float32),
        input_output_aliases={0: 0},
        compiler_params=pltpu.CompilerParams(needs_layout_passes=False),
        scratch_types=[
            pltpu.VMEM((_RPW,), jnp.int32),
            pltpu.VMEM((_SDEPTH, 16), jnp.float32),
            pltpu.SemaphoreType.DMA((_SDEPTH,)),
        ],
    )(filled, slot_i)
    return out


# speedup vs baseline: 1.0146x; 1.0038x over previous
import jax
import jax.numpy as jnp
from jax import lax
from jax.experimental import pallas as pl
from jax.experimental.pallas import tpu as pltpu
from jax.experimental.pallas import tpu_sc as plsc
from jax._src.pallas import mpmd as _mpmd

_B = 4096
_H = 16384
_NW = 32
_RPW = _B // _NW
_BR = 64
_NCH = _B // _BR
_FDEPTH = 8
_SDEPTH = 8


def _fill_body(slot_hbm, out_hbm, zbuf, sems):
    del slot_hbm
    zbuf[...] = jnp.zeros((_BR, _H), jnp.float32)

    def mk(g):
        return pltpu.make_async_copy(
            zbuf, out_hbm.at[pl.ds(g * _BR, _BR), :], sems.at[g % _FDEPTH]
        )

    for g in range(_NCH):
        if g >= _FDEPTH:
            mk(g - _FDEPTH).wait()
        mk(g).start()
    for g in range(_NCH - _FDEPTH, _NCH):
        mk(g).wait()


def _poke_body(filled_hbm, slot_hbm, out_hbm, slot_v, seg_v, sems):
    del filled_hbm
    nc = 2
    wid = lax.axis_index("s") * nc + lax.axis_index("c")
    base = wid * _RPW

    pltpu.sync_copy(slot_hbm.at[pl.ds(base, _RPW)], slot_v)

    lane = lax.iota(jnp.int32, 16)

    def group(g, _):
        sv = slot_v[pl.ds(g * 16, 16)]
        for k in range(16):
            d = k % _SDEPTH
            r = g * 16 + k
            s = sv[k]
            seg = lax.div(s, 16)
            within = s - seg * 16

            def _wait_prev():
                pltpu.make_async_copy(
                    seg_v.at[d],
                    out_hbm.at[base + r].at[pl.ds(seg * 16, 16)],
                    sems.at[d],
                ).wait()

            if k >= _SDEPTH:
                _wait_prev()
            else:
                pl.when(g > 0)(_wait_prev)

            seg_v[d] = (lane == within).astype(jnp.float32)
            pltpu.make_async_copy(
                seg_v.at[d],
                out_hbm.at[base + r].at[pl.ds(seg * 16, 16)],
                sems.at[d],
            ).start()
        return 0

    lax.fori_loop(0, _RPW // 16, group, 0)
    for d in range(_SDEPTH):
        pltpu.make_async_copy(
            seg_v.at[d], out_hbm.at[base].at[pl.ds(0, 16)], sems.at[d]
        ).wait()


def kernel(hidden_activation, slot_i):
    b, h = hidden_activation.shape
    filled = pl.pallas_call(
        _fill_body,
        in_specs=[pl.BlockSpec(memory_space=pltpu.MemorySpace.HBM)],
        out_specs=pl.BlockSpec(memory_space=pltpu.MemorySpace.HBM),
        out_shape=jax.ShapeDtypeStruct((b, h), jnp.float32),
        scratch_shapes=[
            pltpu.VMEM((_BR, _H), jnp.float32),
            pltpu.SemaphoreType.DMA((_FDEPTH,)),
        ],
    )(slot_i)

    mesh = plsc.VectorSubcoreMesh(core_axis_name="c", subcore_axis_name="s")
    out = _mpmd._mpmd_map(
        [(mesh, _poke_body)],
        jax.ShapeDtypeStruct((b, h), jnp.float32),
        input_output_aliases={0: 0},
        compiler_params=pltpu.CompilerParams(needs_layout_passes=False),
        scratch_types=[
            pltpu.VMEM((_RPW,), jnp.int32),
            pltpu.VMEM((_SDEPTH, 16), jnp.float32),
            pltpu.SemaphoreType.DMA((_SDEPTH,)),
        ],
    )(filled, slot_i)
    return out
